# baseline (speedup 1.0000x reference)
import jax
import jax.numpy as jnp
from jax import lax
from jax.experimental import pallas as pl
from jax.experimental.pallas import tpu as pltpu
from jax.experimental.pallas import tpu_sc as plsc

B = 4096
D = 32
NTILES = 16
CHUNK = B // NTILES
GROUPS = CHUNK // 16

LN2 = 0.6931471805599453
LN100 = 4.605170185988091
X_MAX_POW = 0.75


def _ln(x):
    bits = plsc.bitcast(x, jnp.int32)
    e = lax.shift_right_logical(bits, 23) & 0xFF
    e = e - 127
    m_bits = (bits & 0x007FFFFF) | 0x3F800000
    m = plsc.bitcast(m_bits, jnp.float32)
    big = m >= 1.4142135623730951
    m = jnp.where(big, m * 0.5, m)
    e = e + jnp.where(big, 1, 0)
    s = (m - 1.0) / (m + 1.0)
    s2 = s * s
    lnm = 2.0 * s * (1.0 + s2 * (1.0 / 3.0 + s2 * (0.2 + s2 * (1.0 / 7.0))))
    return LN2 * e.astype(jnp.float32) + lnm


def _glove_kernel(v_hbm, w_hbm, vb_hbm, wb_hbm, c_hbm, i1_hbm, i2_hbm, out_hbm,
                  i1_v, i2_v, vrows, wrows, vt, wt_t, vb_v, wb_v, c_v,
                  stage, shared, red_v, out_stage,
                  sem_v, sem_w, sem_vb, sem_wb):
    sid = lax.axis_index("s")
    base = sid * CHUNK

    pltpu.sync_copy(i1_hbm.at[pl.ds(base, CHUNK)], i1_v)
    pltpu.sync_copy(i2_hbm.at[pl.ds(base, CHUNK)], i2_v)
    pltpu.sync_copy(c_hbm.at[pl.ds(base, CHUNK)], c_v)

    cp_v = pltpu.async_copy(v_hbm.at[i1_v], vrows, sem_v)
    cp_w = pltpu.async_copy(w_hbm.at[i2_v], wrows, sem_w)
    cp_vb = pltpu.async_copy(vb_hbm.at[i1_v], vb_v, sem_vb)
    cp_wb = pltpu.async_copy(wb_hbm.at[i2_v], wb_v, sem_wb)
    cp_v.wait()
    cp_w.wait()
    cp_vb.wait()
    cp_wb.wait()

    iota = lax.broadcasted_iota(jnp.int32, (16,), 0)
    zero = jnp.zeros((16,), jnp.float32)

    idx_lo = iota * CHUNK
    idx_hi = idx_lo + 16 * CHUNK

    def _transpose_body(b, _):
        plsc.store_scatter(vt, [idx_lo + b], vrows[b, pl.ds(0, 16)])
        plsc.store_scatter(vt, [idx_hi + b], vrows[b, pl.ds(16, 16)])
        plsc.store_scatter(wt_t, [idx_lo + b], wrows[b, pl.ds(0, 16)])
        plsc.store_scatter(wt_t, [idx_hi + b], wrows[b, pl.ds(16, 16)])
        return 0

    lax.fori_loop(0, CHUNK, _transpose_body, 0)

    s1 = zero
    s2 = zero
    s3 = zero
    t1 = zero
    t2 = zero
    for g in range(GROUPS):
        gbase = g * 16
        dotv = zero
        for d in range(D):
            off = d * CHUNK + gbase
            dotv = dotv + vt[pl.ds(off, 16)] * wt_t[pl.ds(off, 16)]
        c = c_v[pl.ds(gbase, 16)]
        lnc = _ln(c)
        a = dotv + lnc
        wt = jnp.minimum(jnp.exp(X_MAX_POW * (lnc - LN100)), 1.0)
        s1 = s1 + wt * a * a
        s2 = s2 + wt * a
        s3 = s3 + wt
        bb = vb_v[pl.ds(gbase, 16)] + wb_v[pl.ds(gbase, 16)]
        t1 = t1 + bb
        t2 = t2 + bb * bb

    stage[pl.ds(0, 16)] = s1
    stage[pl.ds(16, 16)] = s2
    stage[pl.ds(32, 16)] = s3
    stage[pl.ds(48, 16)] = t1
    stage[pl.ds(64, 16)] = t2
    pltpu.sync_copy(stage, shared.at[sid])
    plsc.subcore_barrier()

    @pl.when(sid == 0)
    def _():
        pltpu.sync_copy(shared, red_v)
        a1 = zero
        a2 = zero
        a3 = zero
        a4 = zero
        a5 = zero
        for t in range(NTILES):
            a1 = a1 + red_v[t, pl.ds(0, 16)]
            a2 = a2 + red_v[t, pl.ds(16, 16)]
            a3 = a3 + red_v[t, pl.ds(32, 16)]
            a4 = a4 + red_v[t, pl.ds(48, 16)]
            a5 = a5 + red_v[t, pl.ds(64, 16)]
        S1 = jnp.sum(a1)
        S2 = jnp.sum(a2)
        S3 = jnp.sum(a3)
        T1 = jnp.sum(a4)
        T2 = jnp.sum(a5)
        final = float(B) * S1 + 2.0 * S2 * T1 + S3 * T2
        out_stage[...] = jnp.full((16,), final, jnp.float32)
        pltpu.sync_copy(out_stage, out_hbm)


def kernel(v_table, w_table, v_bias_table, w_bias_table, cooccur,
           feature_idx1, feature_idx2):
    mesh = plsc.VectorSubcoreMesh(
        core_axis_name="c", subcore_axis_name="s", num_cores=1)
    run = pl.kernel(
        _glove_kernel,
        out_type=jax.ShapeDtypeStruct((16,), jnp.float32),
        mesh=mesh,
        compiler_params=pltpu.CompilerParams(
            needs_layout_passes=False, use_tc_tiling_on_sc=False),
        scratch_types=[
            pltpu.VMEM((CHUNK,), jnp.int32),
            pltpu.VMEM((CHUNK,), jnp.int32),
            pltpu.VMEM((CHUNK, D), jnp.float32),
            pltpu.VMEM((CHUNK, D), jnp.float32),
            pltpu.VMEM((CHUNK * D,), jnp.float32),
            pltpu.VMEM((CHUNK * D,), jnp.float32),
            pltpu.VMEM((CHUNK,), jnp.float32),
            pltpu.VMEM((CHUNK,), jnp.float32),
            pltpu.VMEM((CHUNK,), jnp.float32),
            pltpu.VMEM((80,), jnp.float32),
            pltpu.VMEM_SHARED((NTILES, 80), jnp.float32),
            pltpu.VMEM((NTILES, 80), jnp.float32),
            pltpu.VMEM((16,), jnp.float32),
            pltpu.SemaphoreType.DMA,
            pltpu.SemaphoreType.DMA,
            pltpu.SemaphoreType.DMA,
            pltpu.SemaphoreType.DMA,
        ],
    )
    out = run(v_table, w_table,
              v_bias_table.reshape(-1), w_bias_table.reshape(-1),
              cooccur, feature_idx1.astype(jnp.int32),
              feature_idx2.astype(jnp.int32))
    return out[0]
